# baseline (speedup 1.0000x reference)
import functools

import jax
import jax.numpy as jnp
from jax import lax
from jax.experimental import pallas as pl
from jax.experimental.pallas import tpu as pltpu
from jax.experimental.pallas import tpu_sc as plsc

ALPHA = 1.0
LAMBD = 1.0
K_ITERS = 8

NW = 32
CHUNK = 128


def _mesh():
    return plsc.VectorSubcoreMesh(core_axis_name="c", subcore_axis_name="s",
                                  num_cores=2, num_subcores=16)


def _make_sc_propagate(n_pad, d, n_chunks):
    rows_per_sub = n_pad // 16

    @functools.partial(
        pl.kernel,
        mesh=_mesh(),
        compiler_params=pltpu.CompilerParams(use_tc_tiling_on_sc=False),
        out_type=jax.ShapeDtypeStruct((2, n_pad, d), jnp.float32),
        scratch_types=[
            pltpu.VMEM((n_chunks, CHUNK), jnp.int32),
            pltpu.VMEM((n_chunks, CHUNK), jnp.int32),
            [pltpu.VMEM((CHUNK, d), jnp.float32) for _ in range(6)],
            pltpu.VMEM_SHARED((n_pad, d), jnp.float32),
            pltpu.SemaphoreType.DMA,
            pltpu.SemaphoreType.DMA,
        ],
    )
    def prop(g_hbm, src_hbm, dst_hbm, zeros_hbm, out_hbm,
             src_v, dst_v, bufs, acc_sh, sem_g, sem_s):
        c = lax.axis_index("c")
        s = lax.axis_index("s")
        wid = s * 2 + c

        pltpu.sync_copy(src_hbm.at[wid], src_v)
        pltpu.sync_copy(dst_hbm.at[wid], dst_v)
        lo = s * rows_per_sub
        pltpu.sync_copy(zeros_hbm.at[pl.ds(lo, rows_per_sub)],
                        acc_sh.at[pl.ds(lo, rows_per_sub)])
        plsc.subcore_barrier()

        def fire_g(j, buf):
            pltpu.async_copy(g_hbm.at[src_v.at[j]], buf, sem_g)

        def wait_g(buf):
            pltpu.make_async_copy(g_hbm.at[src_v.at[0]], buf, sem_g).wait()

        def fire_s(j, buf):
            pltpu.async_copy(buf, acc_sh.at[dst_v.at[j]], sem_s, add=True)

        def wait_s(buf):
            pltpu.make_async_copy(buf, acc_sh.at[dst_v.at[0]], sem_s).wait()

        def waits_g(grp):
            for b in grp:
                wait_g(b)

        def waits_s(grp):
            for b in grp:
                wait_s(b)

        def fires_g(base, grp):
            for i in range(3):
                fire_g(base + i, grp[i])

        def fires_s(base, grp):
            for i in range(3):
                fire_s(base + i, grp[i])

        x_grp, y_grp = bufs[0:3], bufs[3:6]
        fires_g(0, x_grp)
        waits_g(x_grp); fires_s(0, x_grp); fires_g(3, y_grp)
        waits_g(y_grp); fires_s(3, y_grp); waits_s(x_grp); fires_g(6, x_grp)

        def body(u, _):
            b0 = 6 * u
            waits_g(x_grp); fires_s(b0, x_grp); waits_s(y_grp); fires_g(b0 + 3, y_grp)
            waits_g(y_grp); fires_s(b0 + 3, y_grp); waits_s(x_grp); fires_g(b0 + 6, x_grp)
            return _

        lax.fori_loop(1, n_chunks // 6 - 1, body, 0)
        b0 = n_chunks - 6
        waits_g(x_grp); fires_s(b0, x_grp); waits_s(y_grp); fires_g(b0 + 3, y_grp)
        waits_g(y_grp); fires_s(b0 + 3, y_grp); waits_s(x_grp)
        waits_s(y_grp)

        plsc.subcore_barrier()
        pltpu.sync_copy(acc_sh.at[pl.ds(lo, rows_per_sub)],
                        out_hbm.at[c, pl.ds(lo, rows_per_sub)])

    return prop


def _dinv_body(deg_ref, dinv_ref, dinv2_ref):
    dsum = deg_ref[0] + deg_ref[1]
    di = lax.rsqrt(jnp.maximum(dsum[:, 0:1], 1.0))
    dinv_ref[...] = di
    dinv2_ref[...] = di * di


def _dinv_call(deg_p, n):
    blk = n // 10
    return pl.pallas_call(
        _dinv_body,
        grid=(10,),
        in_specs=[pl.BlockSpec((2, blk, 64), lambda i: (0, i, 0))],
        out_specs=[pl.BlockSpec((blk, 1), lambda i: (i, 0)),
                   pl.BlockSpec((blk, 1), lambda i: (i, 0))],
        out_shape=[jax.ShapeDtypeStruct((n, 1), jnp.float32),
                   jax.ShapeDtypeStruct((n, 1), jnp.float32)],
    )(deg_p)


def _mlp_body(x_ref, w_ref, b_ref, dinv_ref, h_ref, hs_ref):
    acc = jnp.dot(x_ref[...], w_ref[...], preferred_element_type=jnp.float32)
    h = jnp.maximum(acc + b_ref[...], 0.0)
    h_ref[...] = h
    hs_ref[...] = h * dinv_ref[...]


def _mlp_call(x, w, b, dinv):
    n, f_in = x.shape
    f_out = w.shape[1]
    blk = n // 10
    return pl.pallas_call(
        _mlp_body,
        grid=(10,),
        in_specs=[pl.BlockSpec((blk, f_in), lambda i: (i, 0)),
                  pl.BlockSpec((f_in, f_out), lambda i: (0, 0)),
                  pl.BlockSpec((1, f_out), lambda i: (0, 0)),
                  pl.BlockSpec((blk, 1), lambda i: (i, 0))],
        out_specs=[pl.BlockSpec((blk, f_out), lambda i: (i, 0)),
                   pl.BlockSpec((blk, f_out), lambda i: (i, 0))],
        out_shape=[jax.ShapeDtypeStruct((n, f_out), jnp.float32),
                   jax.ShapeDtypeStruct((n, f_out), jnp.float32)],
    )(x, w, b.reshape(1, f_out), dinv)


def _combine_body(acc_ref, h0_ref, h0s_ref, dinv_ref, dinv2_ref, g_ref, h_ref):
    ssum = acc_ref[0] + acc_ref[1]
    g_ref[...] = 0.5 * (h0s_ref[...] + dinv2_ref[...] * ssum)
    h_ref[...] = 0.5 * (h0_ref[...] + dinv_ref[...] * ssum)


def _combine_call(acc_p, h0, h0s, dinv, dinv2):
    n, d = h0.shape
    blk = n // 10
    return pl.pallas_call(
        _combine_body,
        grid=(10,),
        in_specs=[pl.BlockSpec((2, blk, d), lambda i: (0, i, 0)),
                  pl.BlockSpec((blk, d), lambda i: (i, 0)),
                  pl.BlockSpec((blk, d), lambda i: (i, 0)),
                  pl.BlockSpec((blk, 1), lambda i: (i, 0)),
                  pl.BlockSpec((blk, 1), lambda i: (i, 0))],
        out_specs=[pl.BlockSpec((blk, d), lambda i: (i, 0)),
                   pl.BlockSpec((blk, d), lambda i: (i, 0))],
        out_shape=[jax.ShapeDtypeStruct((n, d), jnp.float32),
                   jax.ShapeDtypeStruct((n, d), jnp.float32)],
    )(acc_p, h0, h0s, dinv, dinv2)


def kernel(graph, features, W1, b1, W2, b2):
    n, _ = features.shape
    e = graph.shape[1]
    d = W1.shape[1]

    e_tot = e + n
    n_chunks = -(-e_tot // (NW * CHUNK))
    n_chunks = 6 * (-(-n_chunks // 6))
    e_pad = NW * n_chunks * CHUNK
    loop_idx = jnp.arange(n, dtype=jnp.int32)
    pad_src = jnp.zeros((e_pad - e_tot,), dtype=jnp.int32)
    pad_dst = jnp.full((e_pad - e_tot,), n, dtype=jnp.int32)
    src_l = jnp.concatenate([graph[0], loop_idx, pad_src]).reshape(NW, n_chunks, CHUNK)
    dst_l = jnp.concatenate([graph[1], loop_idx, pad_dst]).reshape(NW, n_chunks, CHUNK)

    n_pad = 128 * (-(-(n + 1) // 128))
    zeros_d = jnp.zeros((n_pad, d), dtype=jnp.float32)
    ones_nd = jnp.ones((n, d), dtype=jnp.float32)

    sc_prop = _make_sc_propagate(n_pad, d, n_chunks)

    deg_p = sc_prop(ones_nd, src_l, dst_l, zeros_d)
    dinv, dinv2 = _dinv_call(deg_p, n)

    h0, h0s = _mlp_call(features, W1, b1, dinv)
    g = h0s
    for _ in range(K_ITERS):
        acc_p = sc_prop(g, src_l, dst_l, zeros_d)
        g, h = _combine_call(acc_p, h0, h0s, dinv, dinv2)
    h2, h2s = _mlp_call(h, W2, b2, dinv)
    g = h2s
    for _ in range(K_ITERS):
        acc_p = sc_prop(g, src_l, dst_l, zeros_d)
        g, h = _combine_call(acc_p, h2, h2s, dinv, dinv2)
    return h


# speedup vs baseline: 1.4325x; 1.4325x over previous
import functools

import jax
import jax.numpy as jnp
from jax import lax
from jax.experimental import pallas as pl
from jax.experimental.pallas import tpu as pltpu
from jax.experimental.pallas import tpu_sc as plsc

ALPHA = 1.0
LAMBD = 1.0
K_ITERS = 8

NW = 32
CHUNK = 128


def _mesh():
    return plsc.VectorSubcoreMesh(core_axis_name="c", subcore_axis_name="s",
                                  num_cores=2, num_subcores=16)


def _make_sc_propagate(n_pad, d, n_chunks):
    rows_per_sub = n_pad // 16

    @functools.partial(
        pl.kernel,
        mesh=_mesh(),
        compiler_params=pltpu.CompilerParams(use_tc_tiling_on_sc=False),
        out_type=jax.ShapeDtypeStruct((2, n_pad, d), jnp.float32),
        scratch_types=[
            pltpu.VMEM((n_chunks, CHUNK), jnp.int32),
            pltpu.VMEM((n_chunks, CHUNK), jnp.int32),
            [pltpu.VMEM((CHUNK, d), jnp.float32) for _ in range(4)],
            pltpu.VMEM_SHARED((n_pad, d), jnp.float32),
            pltpu.VMEM_SHARED((n_pad, d), jnp.float32),
            pltpu.SemaphoreType.DMA,
            pltpu.SemaphoreType.DMA,
        ],
    )
    def prop(g_hbm, src_hbm, dst_hbm, zeros_hbm, out_hbm,
             src_v, dst_v, bufs, acc_sh, g_sh, sem_g, sem_s):
        c = lax.axis_index("c")
        s = lax.axis_index("s")
        wid = s * 2 + c

        pltpu.sync_copy(src_hbm.at[wid], src_v)
        pltpu.sync_copy(dst_hbm.at[wid], dst_v)
        lo = s * rows_per_sub
        pltpu.sync_copy(zeros_hbm.at[pl.ds(lo, rows_per_sub)],
                        acc_sh.at[pl.ds(lo, rows_per_sub)])
        n_stage = g_hbm.shape[0]
        chunk_g = -(-n_stage // 16) // 8 * 8 + 8
        last_g = n_stage - 15 * chunk_g

        @pl.when(s < 15)
        def _():
            pltpu.sync_copy(g_hbm.at[pl.ds(s * chunk_g, chunk_g)],
                            g_sh.at[pl.ds(s * chunk_g, chunk_g)])

        @pl.when(s == 15)
        def _():
            pltpu.sync_copy(g_hbm.at[pl.ds(15 * chunk_g, last_g)],
                            g_sh.at[pl.ds(15 * chunk_g, last_g)])

        plsc.subcore_barrier()

        def fire_g(j, buf):
            pltpu.async_copy(g_sh.at[src_v.at[j]], buf, sem_g)

        def wait_g(buf):
            pltpu.make_async_copy(g_sh.at[src_v.at[0]], buf, sem_g).wait()

        def fire_s(j, buf):
            pltpu.async_copy(buf, acc_sh.at[dst_v.at[j]], sem_s, add=True)

        def wait_s(buf):
            pltpu.make_async_copy(buf, acc_sh.at[dst_v.at[0]], sem_s).wait()

        def waits_g(grp):
            for b in grp:
                wait_g(b)

        def waits_s(grp):
            for b in grp:
                wait_s(b)

        def fires_g(base, grp):
            for i, b in enumerate(grp):
                fire_g(base + i, b)

        def fires_s(base, grp):
            for i, b in enumerate(grp):
                fire_s(base + i, b)

        x_grp, y_grp = bufs[0:2], bufs[2:4]
        fires_g(0, x_grp)
        waits_g(x_grp); fires_s(0, x_grp); fires_g(2, y_grp)
        waits_g(y_grp); fires_s(2, y_grp); waits_s(x_grp); fires_g(4, x_grp)

        def body(u, _):
            b0 = 4 * u
            waits_g(x_grp); fires_s(b0, x_grp); waits_s(y_grp); fires_g(b0 + 2, y_grp)
            waits_g(y_grp); fires_s(b0 + 2, y_grp); waits_s(x_grp); fires_g(b0 + 4, x_grp)
            return _

        lax.fori_loop(1, n_chunks // 4 - 1, body, 0)
        b0 = n_chunks - 4
        waits_g(x_grp); fires_s(b0, x_grp); waits_s(y_grp); fires_g(b0 + 2, y_grp)
        waits_g(y_grp); fires_s(b0 + 2, y_grp); waits_s(x_grp)
        waits_s(y_grp)

        plsc.subcore_barrier()
        pltpu.sync_copy(acc_sh.at[pl.ds(lo, rows_per_sub)],
                        out_hbm.at[c, pl.ds(lo, rows_per_sub)])

    return prop


def _dinv_body(deg_ref, dinv_ref, dinv2_ref):
    dsum = deg_ref[0] + deg_ref[1]
    di = lax.rsqrt(jnp.maximum(dsum[:, 0:1], 1.0))
    dinv_ref[...] = di
    dinv2_ref[...] = di * di


def _dinv_call(deg_p, n):
    blk = n // 10
    return pl.pallas_call(
        _dinv_body,
        grid=(10,),
        in_specs=[pl.BlockSpec((2, blk, 64), lambda i: (0, i, 0))],
        out_specs=[pl.BlockSpec((blk, 1), lambda i: (i, 0)),
                   pl.BlockSpec((blk, 1), lambda i: (i, 0))],
        out_shape=[jax.ShapeDtypeStruct((n, 1), jnp.float32),
                   jax.ShapeDtypeStruct((n, 1), jnp.float32)],
    )(deg_p)


def _mlp_body(x_ref, w_ref, b_ref, dinv_ref, h_ref, hs_ref):
    acc = jnp.dot(x_ref[...], w_ref[...], preferred_element_type=jnp.float32)
    h = jnp.maximum(acc + b_ref[...], 0.0)
    h_ref[...] = h
    hs_ref[...] = h * dinv_ref[...]


def _mlp_call(x, w, b, dinv):
    n, f_in = x.shape
    f_out = w.shape[1]
    blk = n // 10
    return pl.pallas_call(
        _mlp_body,
        grid=(10,),
        in_specs=[pl.BlockSpec((blk, f_in), lambda i: (i, 0)),
                  pl.BlockSpec((f_in, f_out), lambda i: (0, 0)),
                  pl.BlockSpec((1, f_out), lambda i: (0, 0)),
                  pl.BlockSpec((blk, 1), lambda i: (i, 0))],
        out_specs=[pl.BlockSpec((blk, f_out), lambda i: (i, 0)),
                   pl.BlockSpec((blk, f_out), lambda i: (i, 0))],
        out_shape=[jax.ShapeDtypeStruct((n, f_out), jnp.float32),
                   jax.ShapeDtypeStruct((n, f_out), jnp.float32)],
    )(x, w, b.reshape(1, f_out), dinv)


def _combine_body(acc_ref, h0_ref, h0s_ref, dinv_ref, dinv2_ref, g_ref, h_ref):
    ssum = acc_ref[0] + acc_ref[1]
    g_ref[...] = 0.5 * (h0s_ref[...] + dinv2_ref[...] * ssum)
    h_ref[...] = 0.5 * (h0_ref[...] + dinv_ref[...] * ssum)


def _combine_call(acc_p, h0, h0s, dinv, dinv2):
    n, d = h0.shape
    blk = n // 10
    return pl.pallas_call(
        _combine_body,
        grid=(10,),
        in_specs=[pl.BlockSpec((2, blk, d), lambda i: (0, i, 0)),
                  pl.BlockSpec((blk, d), lambda i: (i, 0)),
                  pl.BlockSpec((blk, d), lambda i: (i, 0)),
                  pl.BlockSpec((blk, 1), lambda i: (i, 0)),
                  pl.BlockSpec((blk, 1), lambda i: (i, 0))],
        out_specs=[pl.BlockSpec((blk, d), lambda i: (i, 0)),
                   pl.BlockSpec((blk, d), lambda i: (i, 0))],
        out_shape=[jax.ShapeDtypeStruct((n, d), jnp.float32),
                   jax.ShapeDtypeStruct((n, d), jnp.float32)],
    )(acc_p, h0, h0s, dinv, dinv2)


def kernel(graph, features, W1, b1, W2, b2):
    n, _ = features.shape
    e = graph.shape[1]
    d = W1.shape[1]

    e_tot = e + n
    n_chunks = -(-e_tot // (NW * CHUNK))
    n_chunks = 4 * (-(-n_chunks // 4))
    e_pad = NW * n_chunks * CHUNK
    loop_idx = jnp.arange(n, dtype=jnp.int32)
    pad_src = jnp.zeros((e_pad - e_tot,), dtype=jnp.int32)
    pad_dst = jnp.full((e_pad - e_tot,), n, dtype=jnp.int32)
    src_l = jnp.concatenate([graph[0], loop_idx, pad_src]).reshape(NW, n_chunks, CHUNK)
    dst_l = jnp.concatenate([graph[1], loop_idx, pad_dst]).reshape(NW, n_chunks, CHUNK)

    n_pad = 128 * (-(-(n + 1) // 128))
    zeros_d = jnp.zeros((n_pad, d), dtype=jnp.float32)
    ones_nd = jnp.ones((n, d), dtype=jnp.float32)

    sc_prop = _make_sc_propagate(n_pad, d, n_chunks)

    deg_p = sc_prop(ones_nd, src_l, dst_l, zeros_d)
    dinv, dinv2 = _dinv_call(deg_p, n)

    h0, h0s = _mlp_call(features, W1, b1, dinv)
    g = h0s
    for _ in range(K_ITERS):
        acc_p = sc_prop(g, src_l, dst_l, zeros_d)
        g, h = _combine_call(acc_p, h0, h0s, dinv, dinv2)
    h2, h2s = _mlp_call(h, W2, b2, dinv)
    g = h2s
    for _ in range(K_ITERS):
        acc_p = sc_prop(g, src_l, dst_l, zeros_d)
        g, h = _combine_call(acc_p, h2, h2s, dinv, dinv2)
    return h


# speedup vs baseline: 1.5270x; 1.0660x over previous
import functools

import jax
import jax.numpy as jnp
from jax import lax
from jax.experimental import pallas as pl
from jax.experimental.pallas import tpu as pltpu
from jax.experimental.pallas import tpu_sc as plsc

ALPHA = 1.0
LAMBD = 1.0
K_ITERS = 8

NW = 32
CHUNK = 128


def _mesh():
    return plsc.VectorSubcoreMesh(core_axis_name="c", subcore_axis_name="s",
                                  num_cores=2, num_subcores=16)


def _make_sc_propagate(n_pad, d, n_chunks):
    rows_per_sub = n_pad // 16

    @functools.partial(
        pl.kernel,
        mesh=_mesh(),
        compiler_params=pltpu.CompilerParams(use_tc_tiling_on_sc=False),
        out_type=[jax.ShapeDtypeStruct((n_pad, d), jnp.float32),
                  jax.ShapeDtypeStruct((n_pad, d), jnp.float32)],
        scratch_types=[
            pltpu.VMEM((n_chunks, CHUNK), jnp.int32),
            pltpu.VMEM((n_chunks, CHUNK), jnp.int32),
            [pltpu.VMEM((CHUNK, d), jnp.float32) for _ in range(4)],
            pltpu.VMEM_SHARED((n_pad, d), jnp.float32),
            pltpu.VMEM_SHARED((n_pad, d), jnp.float32),
            pltpu.SemaphoreType.DMA,
            pltpu.SemaphoreType.DMA,
        ],
    )
    def prop(g_hbm, src_hbm, dst_hbm, zeros_hbm, out0_hbm, out1_hbm,
             src_v, dst_v, bufs, acc_sh, g_sh, sem_g, sem_s):
        c = lax.axis_index("c")
        s = lax.axis_index("s")
        wid = s * 2 + c

        pltpu.sync_copy(src_hbm.at[wid], src_v)
        pltpu.sync_copy(dst_hbm.at[wid], dst_v)
        lo = s * rows_per_sub
        pltpu.sync_copy(zeros_hbm.at[pl.ds(lo, rows_per_sub)],
                        acc_sh.at[pl.ds(lo, rows_per_sub)])
        n_stage = g_hbm.shape[0]
        chunk_g = -(-n_stage // 16) // 8 * 8 + 8
        last_g = n_stage - 15 * chunk_g

        @pl.when(s < 15)
        def _():
            pltpu.sync_copy(g_hbm.at[pl.ds(s * chunk_g, chunk_g)],
                            g_sh.at[pl.ds(s * chunk_g, chunk_g)])

        @pl.when(s == 15)
        def _():
            pltpu.sync_copy(g_hbm.at[pl.ds(15 * chunk_g, last_g)],
                            g_sh.at[pl.ds(15 * chunk_g, last_g)])

        plsc.subcore_barrier()

        def fire_g(j, buf):
            pltpu.async_copy(g_sh.at[src_v.at[j]], buf, sem_g)

        def wait_g(buf):
            pltpu.make_async_copy(g_sh.at[src_v.at[0]], buf, sem_g).wait()

        def fire_s(j, buf):
            pltpu.async_copy(buf, acc_sh.at[dst_v.at[j]], sem_s, add=True)

        def wait_s(buf):
            pltpu.make_async_copy(buf, acc_sh.at[dst_v.at[0]], sem_s).wait()

        def waits_g(grp):
            for b in grp:
                wait_g(b)

        def waits_s(grp):
            for b in grp:
                wait_s(b)

        def fires_g(base, grp):
            for i, b in enumerate(grp):
                fire_g(base + i, b)

        def fires_s(base, grp):
            for i, b in enumerate(grp):
                fire_s(base + i, b)

        x_grp, y_grp = bufs[0:2], bufs[2:4]
        fires_g(0, x_grp)
        waits_g(x_grp); fires_s(0, x_grp); fires_g(2, y_grp)
        waits_g(y_grp); fires_s(2, y_grp); waits_s(x_grp); fires_g(4, x_grp)

        def body(u, _):
            b0 = 4 * u
            waits_g(x_grp); fires_s(b0, x_grp); waits_s(y_grp); fires_g(b0 + 2, y_grp)
            waits_g(y_grp); fires_s(b0 + 2, y_grp); waits_s(x_grp); fires_g(b0 + 4, x_grp)
            return _

        lax.fori_loop(1, n_chunks // 4 - 1, body, 0)
        b0 = n_chunks - 4
        waits_g(x_grp); fires_s(b0, x_grp); waits_s(y_grp); fires_g(b0 + 2, y_grp)
        waits_g(y_grp); fires_s(b0 + 2, y_grp); waits_s(x_grp)
        waits_s(y_grp)

        plsc.subcore_barrier()

        @pl.when(c == 0)
        def _():
            pltpu.sync_copy(acc_sh.at[pl.ds(lo, rows_per_sub)],
                            out0_hbm.at[pl.ds(lo, rows_per_sub)])

        @pl.when(c == 1)
        def _():
            pltpu.sync_copy(acc_sh.at[pl.ds(lo, rows_per_sub)],
                            out1_hbm.at[pl.ds(lo, rows_per_sub)])

    return prop


def _dinv_body(deg0_ref, deg1_ref, dinv_ref, dinv2_ref):
    dsum = deg0_ref[...] + deg1_ref[...]
    di = lax.rsqrt(jnp.maximum(dsum[:, 0:1], 1.0))
    dinv_ref[...] = di
    dinv2_ref[...] = di * di


def _dinv_call(deg0, deg1, n):
    blk = n // 10
    return pl.pallas_call(
        _dinv_body,
        grid=(10,),
        in_specs=[pl.BlockSpec((blk, 64), lambda i: (i, 0)),
                  pl.BlockSpec((blk, 64), lambda i: (i, 0))],
        out_specs=[pl.BlockSpec((blk, 1), lambda i: (i, 0)),
                   pl.BlockSpec((blk, 1), lambda i: (i, 0))],
        out_shape=[jax.ShapeDtypeStruct((n, 1), jnp.float32),
                   jax.ShapeDtypeStruct((n, 1), jnp.float32)],
    )(deg0, deg1)


def _mlp_body(x_ref, w_ref, b_ref, dinv_ref, h_ref, hs_ref):
    acc = jnp.dot(x_ref[...], w_ref[...], preferred_element_type=jnp.float32)
    h = jnp.maximum(acc + b_ref[...], 0.0)
    h_ref[...] = h
    hs_ref[...] = h * dinv_ref[...]


def _mlp_call(x, w, b, dinv):
    n, f_in = x.shape
    f_out = w.shape[1]
    blk = n // 10
    return pl.pallas_call(
        _mlp_body,
        grid=(10,),
        in_specs=[pl.BlockSpec((blk, f_in), lambda i: (i, 0)),
                  pl.BlockSpec((f_in, f_out), lambda i: (0, 0)),
                  pl.BlockSpec((1, f_out), lambda i: (0, 0)),
                  pl.BlockSpec((blk, 1), lambda i: (i, 0))],
        out_specs=[pl.BlockSpec((blk, f_out), lambda i: (i, 0)),
                   pl.BlockSpec((blk, f_out), lambda i: (i, 0))],
        out_shape=[jax.ShapeDtypeStruct((n, f_out), jnp.float32),
                   jax.ShapeDtypeStruct((n, f_out), jnp.float32)],
    )(x, w, b.reshape(1, f_out), dinv)


def _combine_g_body(a0_ref, a1_ref, h0s_ref, dinv2_ref, g_ref):
    ssum = a0_ref[...] + a1_ref[...]
    g_ref[...] = 0.5 * (h0s_ref[...] + dinv2_ref[...] * ssum)


def _combine_g_call(a0, a1, h0s, dinv2):
    n, d = h0s.shape
    blk = n // 10
    return pl.pallas_call(
        _combine_g_body,
        grid=(10,),
        in_specs=[pl.BlockSpec((blk, d), lambda i: (i, 0)),
                  pl.BlockSpec((blk, d), lambda i: (i, 0)),
                  pl.BlockSpec((blk, d), lambda i: (i, 0)),
                  pl.BlockSpec((blk, 1), lambda i: (i, 0))],
        out_specs=pl.BlockSpec((blk, d), lambda i: (i, 0)),
        out_shape=jax.ShapeDtypeStruct((n, d), jnp.float32),
    )(a0, a1, h0s, dinv2)


def _combine_h_body(a0_ref, a1_ref, h0_ref, dinv_ref, h_ref):
    ssum = a0_ref[...] + a1_ref[...]
    h_ref[...] = 0.5 * (h0_ref[...] + dinv_ref[...] * ssum)


def _combine_h_call(a0, a1, h0, dinv):
    n, d = h0.shape
    blk = n // 10
    return pl.pallas_call(
        _combine_h_body,
        grid=(10,),
        in_specs=[pl.BlockSpec((blk, d), lambda i: (i, 0)),
                  pl.BlockSpec((blk, d), lambda i: (i, 0)),
                  pl.BlockSpec((blk, d), lambda i: (i, 0)),
                  pl.BlockSpec((blk, 1), lambda i: (i, 0))],
        out_specs=pl.BlockSpec((blk, d), lambda i: (i, 0)),
        out_shape=jax.ShapeDtypeStruct((n, d), jnp.float32),
    )(a0, a1, h0, dinv)


def kernel(graph, features, W1, b1, W2, b2):
    n, _ = features.shape
    e = graph.shape[1]
    d = W1.shape[1]

    e_tot = e + n
    n_chunks = -(-e_tot // (NW * CHUNK))
    n_chunks = 4 * (-(-n_chunks // 4))
    e_pad = NW * n_chunks * CHUNK
    loop_idx = jnp.arange(n, dtype=jnp.int32)
    pad_src = jnp.zeros((e_pad - e_tot,), dtype=jnp.int32)
    pad_dst = jnp.full((e_pad - e_tot,), n, dtype=jnp.int32)
    src_l = jnp.concatenate([graph[0], loop_idx, pad_src]).reshape(NW, n_chunks, CHUNK)
    dst_l = jnp.concatenate([graph[1], loop_idx, pad_dst]).reshape(NW, n_chunks, CHUNK)

    n_pad = 128 * (-(-(n + 1) // 128))
    zeros_d = jnp.zeros((n_pad, d), dtype=jnp.float32)
    ones_nd = jnp.ones((n, d), dtype=jnp.float32)

    sc_prop = _make_sc_propagate(n_pad, d, n_chunks)

    deg0, deg1 = sc_prop(ones_nd, src_l, dst_l, zeros_d)
    dinv, dinv2 = _dinv_call(deg0, deg1, n)

    h0, h0s = _mlp_call(features, W1, b1, dinv)
    g = h0s
    for k in range(K_ITERS):
        a0, a1 = sc_prop(g, src_l, dst_l, zeros_d)
        if k < K_ITERS - 1:
            g = _combine_g_call(a0, a1, h0s, dinv2)
        else:
            h = _combine_h_call(a0, a1, h0, dinv)
    h2, h2s = _mlp_call(h, W2, b2, dinv)
    g = h2s
    for k in range(K_ITERS):
        a0, a1 = sc_prop(g, src_l, dst_l, zeros_d)
        if k < K_ITERS - 1:
            g = _combine_g_call(a0, a1, h2s, dinv2)
        else:
            h = _combine_h_call(a0, a1, h2, dinv)
    return h


# speedup vs baseline: 1.5574x; 1.0199x over previous
import functools

import jax
import jax.numpy as jnp
from jax import lax
from jax.experimental import pallas as pl
from jax.experimental.pallas import tpu as pltpu
from jax.experimental.pallas import tpu_sc as plsc

ALPHA = 1.0
LAMBD = 1.0
K_ITERS = 8

NW = 32
CHUNK = 128


def _mesh():
    return plsc.VectorSubcoreMesh(core_axis_name="c", subcore_axis_name="s",
                                  num_cores=2, num_subcores=16)


def _make_sc_propagate(n_pad, d, n_chunks):
    rows_per_sub = n_pad // 16

    @functools.partial(
        pl.kernel,
        mesh=_mesh(),
        compiler_params=pltpu.CompilerParams(use_tc_tiling_on_sc=False),
        out_type=[jax.ShapeDtypeStruct((n_pad, d), jnp.float32),
                  jax.ShapeDtypeStruct((n_pad, d), jnp.float32)],
        scratch_types=[
            pltpu.VMEM((n_chunks, CHUNK), jnp.int32),
            pltpu.VMEM((n_chunks, CHUNK), jnp.int32),
            [pltpu.VMEM((CHUNK, d), jnp.float32) for _ in range(4)],
            pltpu.VMEM_SHARED((n_pad, d), jnp.float32),
            pltpu.VMEM_SHARED((n_pad, d), jnp.float32),
            pltpu.SemaphoreType.DMA,
            pltpu.SemaphoreType.DMA,
        ],
    )
    def prop(g_hbm, src_hbm, dst_hbm, zeros_hbm, out0_hbm, out1_hbm,
             src_v, dst_v, bufs, acc_sh, g_sh, sem_g, sem_s):
        c = lax.axis_index("c")
        s = lax.axis_index("s")
        wid = s * 2 + c

        pltpu.async_copy(src_hbm.at[wid], src_v, sem_s)
        pltpu.async_copy(dst_hbm.at[wid], dst_v, sem_s)
        lo = s * rows_per_sub
        pltpu.async_copy(zeros_hbm.at[pl.ds(lo, rows_per_sub)],
                         acc_sh.at[pl.ds(lo, rows_per_sub)], sem_s)
        n_stage = g_hbm.shape[0]
        chunk_g = -(-n_stage // 16) // 8 * 8 + 8
        last_g = n_stage - 15 * chunk_g

        @pl.when(s < 15)
        def _():
            pltpu.async_copy(g_hbm.at[pl.ds(s * chunk_g, chunk_g)],
                             g_sh.at[pl.ds(s * chunk_g, chunk_g)], sem_s)
            pltpu.make_async_copy(g_hbm.at[pl.ds(s * chunk_g, chunk_g)],
                                  g_sh.at[pl.ds(s * chunk_g, chunk_g)], sem_s).wait()

        @pl.when(s == 15)
        def _():
            pltpu.async_copy(g_hbm.at[pl.ds(15 * chunk_g, last_g)],
                             g_sh.at[pl.ds(15 * chunk_g, last_g)], sem_s)
            pltpu.make_async_copy(g_hbm.at[pl.ds(15 * chunk_g, last_g)],
                                  g_sh.at[pl.ds(15 * chunk_g, last_g)], sem_s).wait()

        pltpu.make_async_copy(src_hbm.at[wid], src_v, sem_s).wait()
        pltpu.make_async_copy(dst_hbm.at[wid], dst_v, sem_s).wait()
        pltpu.make_async_copy(zeros_hbm.at[pl.ds(lo, rows_per_sub)],
                              acc_sh.at[pl.ds(lo, rows_per_sub)], sem_s).wait()
        plsc.subcore_barrier()

        def fire_g(j, buf):
            pltpu.async_copy(g_sh.at[src_v.at[j]], buf, sem_g)

        def wait_g(buf):
            pltpu.make_async_copy(g_sh.at[src_v.at[0]], buf, sem_g).wait()

        def fire_s(j, buf):
            pltpu.async_copy(buf, acc_sh.at[dst_v.at[j]], sem_s, add=True)

        def wait_s(buf):
            pltpu.make_async_copy(buf, acc_sh.at[dst_v.at[0]], sem_s).wait()

        def waits_g(grp):
            for b in grp:
                wait_g(b)

        def waits_s(grp):
            for b in grp:
                wait_s(b)

        def fires_g(base, grp):
            for i, b in enumerate(grp):
                fire_g(base + i, b)

        def fires_s(base, grp):
            for i, b in enumerate(grp):
                fire_s(base + i, b)

        x_grp, y_grp = bufs[0:2], bufs[2:4]
        fires_g(0, x_grp)
        waits_g(x_grp); fires_s(0, x_grp); fires_g(2, y_grp)
        waits_g(y_grp); fires_s(2, y_grp); waits_s(x_grp); fires_g(4, x_grp)

        def body(u, _):
            b0 = 4 * u
            waits_g(x_grp); fires_s(b0, x_grp); waits_s(y_grp); fires_g(b0 + 2, y_grp)
            waits_g(y_grp); fires_s(b0 + 2, y_grp); waits_s(x_grp); fires_g(b0 + 4, x_grp)
            return _

        lax.fori_loop(1, n_chunks // 4 - 1, body, 0)
        b0 = n_chunks - 4
        waits_g(x_grp); fires_s(b0, x_grp); waits_s(y_grp); fires_g(b0 + 2, y_grp)
        waits_g(y_grp); fires_s(b0 + 2, y_grp); waits_s(x_grp)
        waits_s(y_grp)

        plsc.subcore_barrier()

        @pl.when(c == 0)
        def _():
            pltpu.sync_copy(acc_sh.at[pl.ds(lo, rows_per_sub)],
                            out0_hbm.at[pl.ds(lo, rows_per_sub)])

        @pl.when(c == 1)
        def _():
            pltpu.sync_copy(acc_sh.at[pl.ds(lo, rows_per_sub)],
                            out1_hbm.at[pl.ds(lo, rows_per_sub)])

    return prop


def _dinv_body(deg0_ref, deg1_ref, dinv_ref, dinv2_ref):
    dsum = deg0_ref[...] + deg1_ref[...]
    di = lax.rsqrt(jnp.maximum(dsum[:, 0:1], 1.0))
    dinv_ref[...] = di
    dinv2_ref[...] = di * di


def _dinv_call(deg0, deg1, n):
    blk = n // 10
    return pl.pallas_call(
        _dinv_body,
        grid=(10,),
        in_specs=[pl.BlockSpec((blk, 64), lambda i: (i, 0)),
                  pl.BlockSpec((blk, 64), lambda i: (i, 0))],
        out_specs=[pl.BlockSpec((blk, 1), lambda i: (i, 0)),
                   pl.BlockSpec((blk, 1), lambda i: (i, 0))],
        out_shape=[jax.ShapeDtypeStruct((n, 1), jnp.float32),
                   jax.ShapeDtypeStruct((n, 1), jnp.float32)],
    )(deg0, deg1)


def _mlp_body(x_ref, w_ref, b_ref, dinv_ref, h_ref, hs_ref):
    acc = jnp.dot(x_ref[...], w_ref[...], preferred_element_type=jnp.float32)
    h = jnp.maximum(acc + b_ref[...], 0.0)
    h_ref[...] = h
    hs_ref[...] = h * dinv_ref[...]


def _mlp_call(x, w, b, dinv):
    n, f_in = x.shape
    f_out = w.shape[1]
    blk = n // 10
    return pl.pallas_call(
        _mlp_body,
        grid=(10,),
        in_specs=[pl.BlockSpec((blk, f_in), lambda i: (i, 0)),
                  pl.BlockSpec((f_in, f_out), lambda i: (0, 0)),
                  pl.BlockSpec((1, f_out), lambda i: (0, 0)),
                  pl.BlockSpec((blk, 1), lambda i: (i, 0))],
        out_specs=[pl.BlockSpec((blk, f_out), lambda i: (i, 0)),
                   pl.BlockSpec((blk, f_out), lambda i: (i, 0))],
        out_shape=[jax.ShapeDtypeStruct((n, f_out), jnp.float32),
                   jax.ShapeDtypeStruct((n, f_out), jnp.float32)],
    )(x, w, b.reshape(1, f_out), dinv)


def _combine_g_body(a0_ref, a1_ref, h0s_ref, dinv2_ref, g_ref):
    ssum = a0_ref[...] + a1_ref[...]
    g_ref[...] = 0.5 * (h0s_ref[...] + dinv2_ref[...] * ssum)


def _combine_g_call(a0, a1, h0s, dinv2):
    n, d = h0s.shape
    blk = n // 10
    return pl.pallas_call(
        _combine_g_body,
        grid=(10,),
        in_specs=[pl.BlockSpec((blk, d), lambda i: (i, 0)),
                  pl.BlockSpec((blk, d), lambda i: (i, 0)),
                  pl.BlockSpec((blk, d), lambda i: (i, 0)),
                  pl.BlockSpec((blk, 1), lambda i: (i, 0))],
        out_specs=pl.BlockSpec((blk, d), lambda i: (i, 0)),
        out_shape=jax.ShapeDtypeStruct((n, d), jnp.float32),
    )(a0, a1, h0s, dinv2)


def _combine_h_body(a0_ref, a1_ref, h0_ref, dinv_ref, h_ref):
    ssum = a0_ref[...] + a1_ref[...]
    h_ref[...] = 0.5 * (h0_ref[...] + dinv_ref[...] * ssum)


def _combine_h_call(a0, a1, h0, dinv):
    n, d = h0.shape
    blk = n // 10
    return pl.pallas_call(
        _combine_h_body,
        grid=(10,),
        in_specs=[pl.BlockSpec((blk, d), lambda i: (i, 0)),
                  pl.BlockSpec((blk, d), lambda i: (i, 0)),
                  pl.BlockSpec((blk, d), lambda i: (i, 0)),
                  pl.BlockSpec((blk, 1), lambda i: (i, 0))],
        out_specs=pl.BlockSpec((blk, d), lambda i: (i, 0)),
        out_shape=jax.ShapeDtypeStruct((n, d), jnp.float32),
    )(a0, a1, h0, dinv)


def kernel(graph, features, W1, b1, W2, b2):
    n, _ = features.shape
    e = graph.shape[1]
    d = W1.shape[1]

    e_tot = e + n
    n_chunks = -(-e_tot // (NW * CHUNK))
    n_chunks = 4 * (-(-n_chunks // 4))
    e_pad = NW * n_chunks * CHUNK
    loop_idx = jnp.arange(n, dtype=jnp.int32)
    pad_src = jnp.zeros((e_pad - e_tot,), dtype=jnp.int32)
    pad_dst = jnp.full((e_pad - e_tot,), n, dtype=jnp.int32)
    src_l = jnp.concatenate([graph[0], loop_idx, pad_src]).reshape(NW, n_chunks, CHUNK)
    dst_l = jnp.concatenate([graph[1], loop_idx, pad_dst]).reshape(NW, n_chunks, CHUNK)

    n_pad = 128 * (-(-(n + 1) // 128))
    zeros_d = jnp.zeros((n_pad, d), dtype=jnp.float32)
    ones_nd = jnp.ones((n, d), dtype=jnp.float32)

    sc_prop = _make_sc_propagate(n_pad, d, n_chunks)

    deg0, deg1 = sc_prop(ones_nd, src_l, dst_l, zeros_d)
    dinv, dinv2 = _dinv_call(deg0, deg1, n)

    h0, h0s = _mlp_call(features, W1, b1, dinv)
    g = h0s
    for k in range(K_ITERS):
        a0, a1 = sc_prop(g, src_l, dst_l, zeros_d)
        if k < K_ITERS - 1:
            g = _combine_g_call(a0, a1, h0s, dinv2)
        else:
            h = _combine_h_call(a0, a1, h0, dinv)
    h2, h2s = _mlp_call(h, W2, b2, dinv)
    g = h2s
    for k in range(K_ITERS):
        a0, a1 = sc_prop(g, src_l, dst_l, zeros_d)
        if k < K_ITERS - 1:
            g = _combine_g_call(a0, a1, h2s, dinv2)
        else:
            h = _combine_h_call(a0, a1, h2, dinv)
    return h
